# baseline (speedup 1.0000x reference)
import functools

import jax
import jax.numpy as jnp
from jax import lax
from jax.experimental import pallas as pl
from jax.experimental.pallas import tpu as pltpu
from jax.experimental.pallas import tpu_sc as plsc

_BATCH = 16384
_VCAT = 52
_VATTR = 128
_L = 16
_NC = 2
_NS = 16
_NW = _NC * _NS
_ROWS_PER_W = _BATCH // _NW
_NGRP = _ROWS_PER_W // _L

_TC_BLK = 512


def _sc_body(cat_hbm, cvoc_hbm, cat_out_hbm, cat_v, cout_v, cvoc_v, sem_in):
    wid = lax.axis_index("s") * _NC + lax.axis_index("c")
    base = wid * _ROWS_PER_W

    pltpu.sync_copy(cvoc_hbm, cvoc_v)
    in_h = pltpu.async_copy(cat_hbm.at[pl.ds(base, _ROWS_PER_W)], cat_v, sem_in)

    lanes = lax.iota(jnp.int32, _L)
    in_h.wait()

    def group_body(g, carry):
        r = g * _L + lanes
        half = _VCAT // 2
        chains = []
        for lo, hi in ((0, half), (half, _VCAT)):
            col0 = jnp.full((_L,), lo, jnp.int32)
            best_v = plsc.load_gather(cat_v, [r, col0])
            best_i = col0
            for j in range(lo + 1, hi):
                col = jnp.full((_L,), j, jnp.int32)
                v = plsc.load_gather(cat_v, [r, col])
                m = v > best_v
                best_v = jnp.where(m, v, best_v)
                best_i = jnp.where(m, col, best_i)
            chains.append((best_v, best_i))
        (v1, i1), (v2, i2) = chains
        m = v2 > v1
        best_i = jnp.where(m, i2, i1)
        cout_v[pl.ds(g * _L, _L)] = plsc.load_gather(cvoc_v, [best_i])
        return carry

    lax.fori_loop(0, _NGRP, group_body, 0)
    pltpu.sync_copy(cout_v, cat_out_hbm.at[pl.ds(base, _ROWS_PER_W)])


_sc_call = functools.partial(
    pl.kernel,
    mesh=plsc.VectorSubcoreMesh(core_axis_name="c", subcore_axis_name="s"),
    compiler_params=pltpu.CompilerParams(needs_layout_passes=False),
    out_type=jax.ShapeDtypeStruct((_BATCH,), jnp.int32),
    scratch_types=[
        pltpu.VMEM((_ROWS_PER_W, _VCAT), jnp.float32),
        pltpu.VMEM((_ROWS_PER_W,), jnp.int32),
        pltpu.VMEM((_VCAT,), jnp.int32),
        pltpu.SemaphoreType.DMA,
    ],
)(_sc_body)


def _tc_body(attr_ref, voc_ref, out_ref):
    a = attr_ref[...]
    voc = voc_ref[...]
    out_ref[...] = jnp.where(a >= 0.5, voc, voc[0, 0])


_tc_call = pl.pallas_call(
    _tc_body,
    grid=(_BATCH // _TC_BLK,),
    in_specs=[
        pl.BlockSpec((_TC_BLK, _VATTR), lambda i: (i, 0)),
        pl.BlockSpec((1, _VATTR), lambda i: (0, 0)),
    ],
    out_specs=pl.BlockSpec((_TC_BLK, _VATTR), lambda i: (i, 0)),
    out_shape=jax.ShapeDtypeStruct((_BATCH, _VATTR), jnp.int32),
)


def kernel(cat_preds, attribute_preds, cat_vocab_ids, attr_vocab_ids):
    cat_out = _sc_call(cat_preds, cat_vocab_ids)
    attr_out = _tc_call(attribute_preds, attr_vocab_ids.reshape(1, _VATTR))
    return cat_out[:, None], attr_out


# speedup vs baseline: 1.0112x; 1.0112x over previous
import functools

import jax
import jax.numpy as jnp
from jax import lax
from jax.experimental import pallas as pl
from jax.experimental.pallas import tpu as pltpu
from jax.experimental.pallas import tpu_sc as plsc

_BATCH = 16384
_VCAT = 52
_VATTR = 128
_L = 16
_NC = 2
_NS = 16
_NW = _NC * _NS
_ROWS_PER_W = _BATCH // _NW
_NGRP = _ROWS_PER_W // _L

_TC_BLK = 2048


def _sc_body(cat_hbm, cvoc_hbm, cat_out_hbm, cat_v, cout_v, cvoc_v, sem_in):
    wid = lax.axis_index("s") * _NC + lax.axis_index("c")
    base = wid * _ROWS_PER_W

    pltpu.sync_copy(cvoc_hbm, cvoc_v)
    in_h = pltpu.async_copy(cat_hbm.at[pl.ds(base, _ROWS_PER_W)], cat_v, sem_in)

    lanes = lax.iota(jnp.int32, _L)
    in_h.wait()

    def group_body(g, carry):
        r = g * _L + lanes
        half = _VCAT // 2
        chains = []
        for lo, hi in ((0, half), (half, _VCAT)):
            col0 = jnp.full((_L,), lo, jnp.int32)
            best_v = plsc.load_gather(cat_v, [r, col0])
            best_i = col0
            for j in range(lo + 1, hi):
                col = jnp.full((_L,), j, jnp.int32)
                v = plsc.load_gather(cat_v, [r, col])
                m = v > best_v
                best_v = jnp.where(m, v, best_v)
                best_i = jnp.where(m, col, best_i)
            chains.append((best_v, best_i))
        (v1, i1), (v2, i2) = chains
        m = v2 > v1
        best_i = jnp.where(m, i2, i1)
        cout_v[pl.ds(g * _L, _L)] = plsc.load_gather(cvoc_v, [best_i])
        return carry

    lax.fori_loop(0, _NGRP, group_body, 0)
    pltpu.sync_copy(cout_v, cat_out_hbm.at[pl.ds(base, _ROWS_PER_W)])


_sc_call = functools.partial(
    pl.kernel,
    mesh=plsc.VectorSubcoreMesh(core_axis_name="c", subcore_axis_name="s"),
    compiler_params=pltpu.CompilerParams(
        needs_layout_passes=False, use_tc_tiling_on_sc=True),
    out_type=jax.ShapeDtypeStruct((_BATCH,), jnp.int32),
    scratch_types=[
        pltpu.VMEM((_ROWS_PER_W, _VCAT), jnp.float32),
        pltpu.VMEM((_ROWS_PER_W,), jnp.int32),
        pltpu.VMEM((_VCAT,), jnp.int32),
        pltpu.SemaphoreType.DMA,
    ],
)(_sc_body)


def _tc_body(attr_ref, voc_ref, out_ref):
    a = attr_ref[...]
    voc = voc_ref[...]
    out_ref[...] = jnp.where(a >= 0.5, voc, voc[0, 0])


_tc_call = pl.pallas_call(
    _tc_body,
    grid=(_BATCH // _TC_BLK,),
    in_specs=[
        pl.BlockSpec((_TC_BLK, _VATTR), lambda i: (i, 0)),
        pl.BlockSpec((1, _VATTR), lambda i: (0, 0)),
    ],
    out_specs=pl.BlockSpec((_TC_BLK, _VATTR), lambda i: (i, 0)),
    out_shape=jax.ShapeDtypeStruct((_BATCH, _VATTR), jnp.int32),
)


def kernel(cat_preds, attribute_preds, cat_vocab_ids, attr_vocab_ids):
    cat_out = _sc_call(cat_preds, cat_vocab_ids)
    attr_out = _tc_call(attribute_preds, attr_vocab_ids.reshape(1, _VATTR))
    return cat_out[:, None], attr_out
